# baseline (speedup 1.0000x reference)
import jax
import jax.numpy as jnp
from jax import lax
from jax.experimental import pallas as pl
from jax.experimental.pallas import tpu as pltpu
from jax.experimental.pallas import tpu_sc as plsc

N = 2048
D_IN = 768
D_OUT = 768
E = 64
T = 128
GL = 128
NCHUNK = 2
RN = N // NCHUNK

NC = 2
NS = 16
NW = NC * NS
BPW = N // NW
CPW = RN // NW


def _sc_mesh():
    return plsc.VectorSubcoreMesh(core_axis_name="c", subcore_axis_name="s",
                                  num_cores=NC, num_subcores=NS)


def _gather_body(r0, inp_hbm, tok_hbm, g2_hbm, xs_hbm, gs_hbm,
                 idx_v, rows_v, rows_g, sem, sem_g):
    wid = lax.axis_index("s") * NC + lax.axis_index("c")
    base = wid * CPW
    pltpu.sync_copy(tok_hbm.at[pl.ds(r0 + base, CPW)], idx_v)
    cp_x = pltpu.async_copy(inp_hbm.at[idx_v], rows_v, sem)
    cp_g = pltpu.async_copy(g2_hbm.at[idx_v], rows_g, sem_g)
    cp_x.wait()
    wb_x = pltpu.async_copy(rows_v, xs_hbm.at[pl.ds(base, CPW)], sem)
    cp_g.wait()
    wb_g = pltpu.async_copy(rows_g, gs_hbm.at[pl.ds(base, CPW)], sem_g)
    wb_x.wait()
    wb_g.wait()


def _sc_gather(inputs, tok, g2, r0):
    body = lambda *a: _gather_body(r0, *a)
    return pl.kernel(
        body,
        out_type=(jax.ShapeDtypeStruct((RN, D_IN), jnp.float32),
                  jax.ShapeDtypeStruct((RN, GL), jnp.float32)),
        mesh=_sc_mesh(),
        scratch_types=[
            pltpu.VMEM((CPW,), jnp.int32),
            pltpu.VMEM((CPW, D_IN), jnp.float32),
            pltpu.VMEM((CPW, GL), jnp.float32),
            pltpu.SemaphoreType.DMA,
            pltpu.SemaphoreType.DMA,
        ],
        name=f"moe_gather_{r0}",
    )(inputs, tok, g2)


def _scatter_half(y_hbm, out_hbm, idx_v, rows_v, sem, lbase):
    pltpu.sync_copy(y_hbm.at[pl.ds(lbase, BPW)], rows_v)
    pltpu.async_copy(rows_v, out_hbm.at[idx_v], sem).wait()


def _scatter_body(y0_hbm, y1_hbm, tok_hbm, out_hbm, idx_v, rows_v, sem):
    wid = lax.axis_index("s") * NC + lax.axis_index("c")
    base = wid * BPW
    pltpu.sync_copy(tok_hbm.at[pl.ds(base, BPW)], idx_v)

    @pl.when(base < RN)
    def _():
        _scatter_half(y0_hbm, out_hbm, idx_v, rows_v, sem, base)

    @pl.when(base >= RN)
    def _():
        _scatter_half(y1_hbm, out_hbm, idx_v, rows_v, sem, base - RN)


def _sc_scatter(y0, y1, tok):
    return pl.kernel(
        _scatter_body,
        out_type=jax.ShapeDtypeStruct((N, D_OUT), jnp.float32),
        mesh=_sc_mesh(),
        scratch_types=[
            pltpu.VMEM((BPW,), jnp.int32),
            pltpu.VMEM((BPW, D_OUT), jnp.float32),
            pltpu.SemaphoreType.DMA,
        ],
        name="moe_scatter",
    )(y0, y1, tok)


NBUF = 4
BLK = 128
NBLK = RN // BLK


def _gemm_body(r0, eb_ref, offs_ref, w_hbm, x_ref, g_ref, y_hbm,
               y_ref, wbuf, sems, sem_y):
    H = D_OUT // 2
    e_lo = eb_ref[0]
    e_hi = eb_ref[1]

    def flush_block(b):
        b = pl.multiple_of(b * BLK, BLK)
        pltpu.make_async_copy(y_ref.at[pl.ds(b, BLK)],
                              y_hbm.at[pl.ds(b, BLK)], sem_y).start()

    def start_fetch(e):
        b = lax.rem(e, NBUF)
        pltpu.make_async_copy(w_hbm.at[e, pl.ds(0, H)],
                              wbuf.at[b, pl.ds(0, H)], sems.at[b, 0]).start()
        pltpu.make_async_copy(w_hbm.at[e, pl.ds(H, H)],
                              wbuf.at[b, pl.ds(H, H)], sems.at[b, 1]).start()

    for j in range(NBUF):
        @pl.when(e_lo + j < e_hi)
        def _():
            start_fetch(e_lo + j)

    def step(e, nb):
        b = lax.rem(e, NBUF)
        pltpu.make_async_copy(w_hbm.at[e, pl.ds(0, H)],
                              wbuf.at[b, pl.ds(0, H)], sems.at[b, 0]).wait()
        pltpu.make_async_copy(w_hbm.at[e, pl.ds(H, H)],
                              wbuf.at[b, pl.ds(H, H)], sems.at[b, 1]).wait()
        s_full = jnp.where(e == 0, 0, offs_ref[jnp.maximum(e - 1, 0)])
        s = jnp.clip(s_full - r0, 0, RN)
        end = jnp.clip(offs_ref[e] - r0, 0, RN)
        s8 = (s // 8) * 8
        nt = (end - s8 + T - 1) // T

        def body(i, _):
            base = pl.multiple_of(jnp.minimum(s8 + i * T, RN - T), 8)
            xg = x_ref[pl.ds(base, T), :] * g_ref[pl.ds(base, T), 0:1]
            y = lax.dot_general(xg, wbuf[b],
                                dimension_numbers=(((1,), (1,)), ((), ())),
                                preferred_element_type=jnp.float32)
            q = base + lax.broadcasted_iota(jnp.int32, (T, D_OUT), 0)
            m = (q >= s) & (q < end)
            y_ref[pl.ds(base, T), :] = jnp.where(m, y,
                                                 y_ref[pl.ds(base, T), :])
            return 0

        lax.fori_loop(0, nt, body, 0)

        @pl.when(e + NBUF < e_hi)
        def _():
            start_fetch(e + NBUF)

        done = end // BLK
        for _ in range(2):
            @pl.when(nb < done)
            def _():
                flush_block(nb)
            nb = jnp.where(nb < done, nb + 1, nb)
        return nb

    nb = lax.fori_loop(e_lo, e_hi, step, 0)
    for i in range(NBLK):
        @pl.when(i >= nb)
        def _():
            flush_block(jnp.int32(i))
    for _ in range(NBLK):
        pltpu.make_async_copy(y_ref.at[pl.ds(0, BLK)],
                              y_hbm.at[pl.ds(0, BLK)], sem_y).wait()


def _tc_grouped_gemm(ebounds, expert_offsets, weight, x_c, g_c, r0):
    body = lambda *a: _gemm_body(r0, *a)
    return pl.pallas_call(
        body,
        in_specs=[
            pl.BlockSpec(memory_space=pltpu.SMEM),
            pl.BlockSpec(memory_space=pltpu.SMEM),
            pl.BlockSpec(memory_space=pltpu.MemorySpace.HBM),
            pl.BlockSpec(memory_space=pltpu.VMEM),
            pl.BlockSpec(memory_space=pltpu.VMEM),
        ],
        out_specs=pl.BlockSpec(memory_space=pltpu.MemorySpace.HBM),
        out_shape=jax.ShapeDtypeStruct((RN, D_OUT), jnp.float32),
        scratch_shapes=[
            pltpu.VMEM((RN, D_OUT), jnp.float32),
            pltpu.VMEM((NBUF, D_OUT, D_IN), jnp.float32),
            pltpu.SemaphoreType.DMA((NBUF, 2)),
            pltpu.SemaphoreType.DMA,
        ],
        name=f"moe_gemm_{r0}",
    )(ebounds, expert_offsets, weight, x_c, g_c)


def kernel(inputs, weight, k, sorted_expert_idxs, sorted_scattered_idxs,
           expert_offsets, gates):
    tok = (sorted_scattered_idxs // k).astype(jnp.int32)
    g2 = jnp.broadcast_to(gates.reshape(N, 1).astype(jnp.float32), (N, GL))
    offs = expert_offsets.astype(jnp.int32)
    ys = []
    for c in range(NCHUNK):
        r0, r1 = c * RN, (c + 1) * RN
        e_lo = jnp.searchsorted(offs, r0, side="right").astype(jnp.int32)
        e_hi = (jnp.searchsorted(offs, r1, side="left") + 1).astype(jnp.int32)
        eb = jnp.stack([e_lo, e_hi])
        x_c, g_c = _sc_gather(inputs, tok, g2, r0)
        ys.append(_tc_grouped_gemm(eb, offs, weight, x_c, g_c, r0))
    return _sc_scatter(ys[0], ys[1], tok)


# speedup vs baseline: 1.0073x; 1.0073x over previous
import jax
import jax.numpy as jnp
from jax import lax
from jax.experimental import pallas as pl
from jax.experimental.pallas import tpu as pltpu
from jax.experimental.pallas import tpu_sc as plsc

N = 2048
D_IN = 768
D_OUT = 768
E = 64
T = 128
GL = 128
NCHUNK = 2
RN = N // NCHUNK

NC = 2
NS = 16
NW = NC * NS
BPW = N // NW
CPW = RN // NW


def _sc_mesh():
    return plsc.VectorSubcoreMesh(core_axis_name="c", subcore_axis_name="s",
                                  num_cores=NC, num_subcores=NS)


def _gather_body(r0, inp_hbm, tok_hbm, g2_hbm, xs_hbm, gs_hbm,
                 idx_v, rows_v, rows_g, sem, sem_g):
    wid = lax.axis_index("s") * NC + lax.axis_index("c")
    base = wid * CPW
    pltpu.sync_copy(tok_hbm.at[pl.ds(r0 + base, CPW)], idx_v)
    cp_x = pltpu.async_copy(inp_hbm.at[idx_v], rows_v, sem)
    cp_g = pltpu.async_copy(g2_hbm.at[idx_v], rows_g, sem_g)
    cp_x.wait()
    wb_x = pltpu.async_copy(rows_v, xs_hbm.at[pl.ds(base, CPW)], sem)
    cp_g.wait()
    wb_g = pltpu.async_copy(rows_g, gs_hbm.at[pl.ds(base, CPW)], sem_g)
    wb_x.wait()
    wb_g.wait()


def _sc_gather(inputs, tok, g2, r0):
    body = lambda *a: _gather_body(r0, *a)
    return pl.kernel(
        body,
        out_type=(jax.ShapeDtypeStruct((RN, D_IN), jnp.float32),
                  jax.ShapeDtypeStruct((RN, GL), jnp.float32)),
        mesh=_sc_mesh(),
        scratch_types=[
            pltpu.VMEM((CPW,), jnp.int32),
            pltpu.VMEM((CPW, D_IN), jnp.float32),
            pltpu.VMEM((CPW, GL), jnp.float32),
            pltpu.SemaphoreType.DMA,
            pltpu.SemaphoreType.DMA,
        ],
        name=f"moe_gather_{r0}",
    )(inputs, tok, g2)


def _scatter_half(y_hbm, out_hbm, idx_v, rows_v, sem, lbase):
    pltpu.sync_copy(y_hbm.at[pl.ds(lbase, BPW)], rows_v)
    pltpu.async_copy(rows_v, out_hbm.at[idx_v], sem).wait()


def _scatter_body(y0_hbm, y1_hbm, tok_hbm, out_hbm, idx_v, rows_v, sem):
    wid = lax.axis_index("s") * NC + lax.axis_index("c")
    base = wid * BPW
    pltpu.sync_copy(tok_hbm.at[pl.ds(base, BPW)], idx_v)

    @pl.when(base < RN)
    def _():
        _scatter_half(y0_hbm, out_hbm, idx_v, rows_v, sem, base)

    @pl.when(base >= RN)
    def _():
        _scatter_half(y1_hbm, out_hbm, idx_v, rows_v, sem, base - RN)


def _sc_scatter(y0, y1, tok):
    return pl.kernel(
        _scatter_body,
        out_type=jax.ShapeDtypeStruct((N, D_OUT), jnp.float32),
        mesh=_sc_mesh(),
        scratch_types=[
            pltpu.VMEM((BPW,), jnp.int32),
            pltpu.VMEM((BPW, D_OUT), jnp.float32),
            pltpu.SemaphoreType.DMA,
        ],
        name="moe_scatter",
    )(y0, y1, tok)


NBUF = 4
BLK = 128
NBLK = RN // BLK


def _gemm_body(r0, eb_ref, offs_ref, w_hbm, x_ref, g_ref, y_hbm,
               y_ref, wbuf, sems, sem_y):
    H = D_OUT // 2
    e_lo = eb_ref[0]
    e_hi = eb_ref[1]

    def flush_block(b):
        b = pl.multiple_of(b * BLK, BLK)
        pltpu.make_async_copy(y_ref.at[pl.ds(b, BLK)],
                              y_hbm.at[pl.ds(b, BLK)], sem_y).start()

    def start_fetch(e):
        b = lax.rem(e, NBUF)
        pltpu.make_async_copy(w_hbm.at[e, pl.ds(0, H)],
                              wbuf.at[b, pl.ds(0, H)], sems.at[b, 0]).start()
        pltpu.make_async_copy(w_hbm.at[e, pl.ds(H, H)],
                              wbuf.at[b, pl.ds(H, H)], sems.at[b, 1]).start()

    for j in range(NBUF):
        @pl.when(e_lo + j < e_hi)
        def _():
            start_fetch(e_lo + j)

    def step(e, nb):
        in_range = (e >= e_lo) & (e < e_hi)
        s_full = jnp.where(e == 0, 0, offs_ref[jnp.maximum(e - 1, 0)])
        s = jnp.clip(s_full - r0, 0, RN)
        end = jnp.clip(offs_ref[e] - r0, 0, RN)

        @pl.when(in_range)
        def _():
            b = lax.rem(e, NBUF)
            pltpu.make_async_copy(w_hbm.at[e, pl.ds(0, H)],
                                  wbuf.at[b, pl.ds(0, H)],
                                  sems.at[b, 0]).wait()
            pltpu.make_async_copy(w_hbm.at[e, pl.ds(H, H)],
                                  wbuf.at[b, pl.ds(H, H)],
                                  sems.at[b, 1]).wait()
            s8 = (s // 8) * 8
            nt = (end - s8 + T - 1) // T

            def body(i, _):
                base = pl.multiple_of(jnp.minimum(s8 + i * T, RN - T), 8)
                xg = x_ref[pl.ds(base, T), :] * g_ref[pl.ds(base, T), 0:1]
                y = lax.dot_general(xg, wbuf[b],
                                    dimension_numbers=(((1,), (1,)), ((), ())),
                                    preferred_element_type=jnp.float32)
                q = base + lax.broadcasted_iota(jnp.int32, (T, D_OUT), 0)
                m = (q >= s) & (q < end)
                y_ref[pl.ds(base, T), :] = jnp.where(m, y,
                                                     y_ref[pl.ds(base, T), :])
                return 0

            lax.fori_loop(0, nt, body, 0)

            @pl.when(e + NBUF < e_hi)
            def _():
                start_fetch(e + NBUF)

        done = jnp.where(e >= e_hi, NBLK, end // BLK)
        for _ in range(2):
            @pl.when(nb < done)
            def _():
                flush_block(nb)
            nb = jnp.where(nb < done, nb + 1, nb)
        return nb

    nb = lax.fori_loop(0, E, step, 0)
    for i in range(NBLK):
        @pl.when(i >= nb)
        def _():
            flush_block(jnp.int32(i))
    for _ in range(NBLK):
        pltpu.make_async_copy(y_ref.at[pl.ds(0, BLK)],
                              y_hbm.at[pl.ds(0, BLK)], sem_y).wait()


def _tc_grouped_gemm(ebounds, expert_offsets, weight, x_c, g_c, r0):
    body = lambda *a: _gemm_body(r0, *a)
    return pl.pallas_call(
        body,
        in_specs=[
            pl.BlockSpec(memory_space=pltpu.SMEM),
            pl.BlockSpec(memory_space=pltpu.SMEM),
            pl.BlockSpec(memory_space=pltpu.MemorySpace.HBM),
            pl.BlockSpec(memory_space=pltpu.VMEM),
            pl.BlockSpec(memory_space=pltpu.VMEM),
        ],
        out_specs=pl.BlockSpec(memory_space=pltpu.MemorySpace.HBM),
        out_shape=jax.ShapeDtypeStruct((RN, D_OUT), jnp.float32),
        scratch_shapes=[
            pltpu.VMEM((RN, D_OUT), jnp.float32),
            pltpu.VMEM((NBUF, D_OUT, D_IN), jnp.float32),
            pltpu.SemaphoreType.DMA((NBUF, 2)),
            pltpu.SemaphoreType.DMA,
        ],
        name=f"moe_gemm_{r0}",
    )(ebounds, expert_offsets, weight, x_c, g_c)


def kernel(inputs, weight, k, sorted_expert_idxs, sorted_scattered_idxs,
           expert_offsets, gates):
    tok = (sorted_scattered_idxs // k).astype(jnp.int32)
    g2 = jnp.broadcast_to(gates.reshape(N, 1).astype(jnp.float32), (N, GL))
    offs = expert_offsets.astype(jnp.int32)
    ys = []
    for c in range(NCHUNK):
        r0, r1 = c * RN, (c + 1) * RN
        e_lo = jnp.searchsorted(offs, r0, side="right").astype(jnp.int32)
        e_hi = (jnp.searchsorted(offs, r1, side="left") + 1).astype(jnp.int32)
        eb = jnp.stack([e_lo, e_hi])
        x_c, g_c = _sc_gather(inputs, tok, g2, r0)
        ys.append(_tc_grouped_gemm(eb, offs, weight, x_c, g_c, r0))
    return _sc_scatter(ys[0], ys[1], tok)


# speedup vs baseline: 1.4609x; 1.4504x over previous
import functools

import jax
import jax.numpy as jnp
from jax import lax
from jax.experimental import pallas as pl
from jax.experimental.pallas import tpu as pltpu
from jax.experimental.pallas import tpu_sc as plsc

N = 2048
D_IN = 768
D_OUT = 768
E = 64
T = 128
GL = 128

NC = 2
NS = 16
NW = NC * NS
BPW = N // NW


def _sc_mesh():
    return plsc.VectorSubcoreMesh(core_axis_name="c", subcore_axis_name="s",
                                  num_cores=NC, num_subcores=NS)


SCH = 4
CW = BPW // SCH


def _gather_body(inp_hbm, tok_hbm, g2_hbm, xs_hbm, gs_hbm,
                 idx_g, rows_g, *rest):
    idx_c = rest[0:SCH]
    rows_c = rest[SCH:2 * SCH]
    sem_st = rest[2 * SCH]
    sem_gx = rest[2 * SCH + 1]
    sem_wb = rest[2 * SCH + 2]
    sem_g = rest[2 * SCH + 3]
    wid = lax.axis_index("s") * NC + lax.axis_index("c")
    base = wid * BPW
    stg = [pltpu.async_copy(tok_hbm.at[pl.ds(base + j * CW, CW)], idx_c[j],
                            sem_st.at[j]) for j in range(SCH)]
    stg_g = pltpu.async_copy(tok_hbm.at[pl.ds(base, BPW)], idx_g, sem_g)
    gx = []
    for j in range(SCH):
        stg[j].wait()
        gx.append(pltpu.async_copy(inp_hbm.at[idx_c[j]], rows_c[j],
                                   sem_gx.at[j]))
    stg_g.wait()
    gg = pltpu.async_copy(g2_hbm.at[idx_g], rows_g, sem_g)
    wb = []
    for j in range(SCH):
        gx[j].wait()
        wb.append(pltpu.async_copy(rows_c[j],
                                   xs_hbm.at[pl.ds(base + j * CW, CW)],
                                   sem_wb.at[j]))
    gg.wait()
    wbg = pltpu.async_copy(rows_g, gs_hbm.at[pl.ds(base, BPW)], sem_g)
    for j in range(SCH):
        wb[j].wait()
    wbg.wait()


def _scatter_body(y_hbm, tok_hbm, out_hbm, *rest):
    idx_c = rest[0:SCH]
    rows_c = rest[SCH:2 * SCH]
    sem_st = rest[2 * SCH]
    sem_ld = rest[2 * SCH + 1]
    sem_sc = rest[2 * SCH + 2]
    wid = lax.axis_index("s") * NC + lax.axis_index("c")
    base = wid * BPW
    stg = [pltpu.async_copy(tok_hbm.at[pl.ds(base + j * CW, CW)], idx_c[j],
                            sem_st.at[j]) for j in range(SCH)]
    ld = [pltpu.async_copy(y_hbm.at[pl.ds(base + j * CW, CW)], rows_c[j],
                           sem_ld.at[j]) for j in range(SCH)]
    sc = []
    for j in range(SCH):
        stg[j].wait()
        ld[j].wait()
        sc.append(pltpu.async_copy(rows_c[j], out_hbm.at[idx_c[j]],
                                   sem_sc.at[j]))
    for j in range(SCH):
        sc[j].wait()


def _sc_gather(inputs, tok, g2):
    return pl.kernel(
        _gather_body,
        out_type=(jax.ShapeDtypeStruct((N, D_IN), jnp.float32),
                  jax.ShapeDtypeStruct((N, GL), jnp.float32)),
        mesh=_sc_mesh(),
        scratch_types=(
            [pltpu.VMEM((BPW,), jnp.int32),
             pltpu.VMEM((BPW, GL), jnp.float32)]
            + [pltpu.VMEM((CW,), jnp.int32) for _ in range(SCH)]
            + [pltpu.VMEM((CW, D_IN), jnp.float32) for _ in range(SCH)]
            + [pltpu.SemaphoreType.DMA((SCH,)),
               pltpu.SemaphoreType.DMA((SCH,)),
               pltpu.SemaphoreType.DMA((SCH,)),
               pltpu.SemaphoreType.DMA]
        ),
    )(inputs, tok, g2)


def _sc_scatter(y_sorted, tok):
    return pl.kernel(
        _scatter_body,
        out_type=jax.ShapeDtypeStruct((N, D_OUT), jnp.float32),
        mesh=_sc_mesh(),
        scratch_types=(
            [pltpu.VMEM((CW,), jnp.int32) for _ in range(SCH)]
            + [pltpu.VMEM((CW, D_OUT), jnp.float32) for _ in range(SCH)]
            + [pltpu.SemaphoreType.DMA((SCH,)),
               pltpu.SemaphoreType.DMA((SCH,)),
               pltpu.SemaphoreType.DMA((SCH,))]
        ),
    )(y_sorted, tok)


NBUF = 4


BLK = 128
NBLK = N // BLK


def _gemm_body(offs_ref, w_hbm, x_ref, g_ref, y_hbm, y_ref, wbuf, sems,
               sem_y):
    H = D_OUT // 2

    def flush_block(b):
        b = pl.multiple_of(b * BLK, BLK)
        pltpu.make_async_copy(y_ref.at[pl.ds(b, BLK)],
                              y_hbm.at[pl.ds(b, BLK)], sem_y).start()

    def start_fetch(e):
        b = lax.rem(e, NBUF)
        pltpu.make_async_copy(w_hbm.at[e, pl.ds(0, H)],
                              wbuf.at[b, pl.ds(0, H)], sems.at[b, 0]).start()
        pltpu.make_async_copy(w_hbm.at[e, pl.ds(H, H)],
                              wbuf.at[b, pl.ds(H, H)], sems.at[b, 1]).start()

    for e in range(NBUF):
        start_fetch(e)

    def step(e, nb):
        b = lax.rem(e, NBUF)
        pltpu.make_async_copy(w_hbm.at[e, pl.ds(0, H)],
                              wbuf.at[b, pl.ds(0, H)], sems.at[b, 0]).wait()
        pltpu.make_async_copy(w_hbm.at[e, pl.ds(H, H)],
                              wbuf.at[b, pl.ds(H, H)], sems.at[b, 1]).wait()
        s = jnp.where(e == 0, 0, offs_ref[jnp.maximum(e - 1, 0)])
        end = offs_ref[e]
        s8 = (s // 8) * 8
        nt = (end - s8 + T - 1) // T

        def body(i, _):
            base = pl.multiple_of(jnp.minimum(s8 + i * T, N - T), 8)
            xg = x_ref[pl.ds(base, T), :] * g_ref[pl.ds(base, T), 0:1]
            y = lax.dot_general(xg, wbuf[b],
                                dimension_numbers=(((1,), (1,)), ((), ())),
                                preferred_element_type=jnp.float32)
            q = base + lax.broadcasted_iota(jnp.int32, (T, D_OUT), 0)
            m = (q >= s) & (q < end)
            y_ref[pl.ds(base, T), :] = jnp.where(m, y,
                                                 y_ref[pl.ds(base, T), :])
            return 0

        lax.fori_loop(0, nt, body, 0)

        @pl.when(e + NBUF < E)
        def _():
            start_fetch(e + NBUF)

        done = end // BLK
        for _ in range(2):
            @pl.when(nb < done)
            def _():
                flush_block(nb)
            nb = jnp.where(nb < done, nb + 1, nb)
        return nb

    nb = lax.fori_loop(0, E, step, 0)
    for i in range(NBLK):
        @pl.when(i >= nb)
        def _():
            flush_block(jnp.int32(i))
    for _ in range(NBLK):
        pltpu.make_async_copy(y_ref.at[pl.ds(0, BLK)],
                              y_hbm.at[pl.ds(0, BLK)], sem_y).wait()


def _tc_grouped_gemm(expert_offsets, weight, x_sorted, g_sorted):
    return pl.pallas_call(
        _gemm_body,
        in_specs=[
            pl.BlockSpec(memory_space=pltpu.SMEM),
            pl.BlockSpec(memory_space=pltpu.MemorySpace.HBM),
            pl.BlockSpec(memory_space=pltpu.VMEM),
            pl.BlockSpec(memory_space=pltpu.VMEM),
        ],
        out_specs=pl.BlockSpec(memory_space=pltpu.MemorySpace.HBM),
        out_shape=jax.ShapeDtypeStruct((N, D_OUT), jnp.float32),
        scratch_shapes=[
            pltpu.VMEM((N, D_OUT), jnp.float32),
            pltpu.VMEM((NBUF, D_OUT, D_IN), jnp.float32),
            pltpu.SemaphoreType.DMA((NBUF, 2)),
            pltpu.SemaphoreType.DMA,
        ],
    )(expert_offsets, weight, x_sorted, g_sorted)


def kernel(inputs, weight, k, sorted_expert_idxs, sorted_scattered_idxs,
           expert_offsets, gates):
    tok = (sorted_scattered_idxs // k).astype(jnp.int32)
    g2 = jnp.broadcast_to(gates.reshape(N, 1).astype(jnp.float32), (N, GL))
    x_sorted, g_sorted = _sc_gather(inputs, tok, g2)
    y_sorted = _tc_grouped_gemm(expert_offsets, weight, x_sorted, g_sorted)
    return _sc_scatter(y_sorted, tok)


# speedup vs baseline: 1.4626x; 1.0011x over previous
import functools

import jax
import jax.numpy as jnp
from jax import lax
from jax.experimental import pallas as pl
from jax.experimental.pallas import tpu as pltpu
from jax.experimental.pallas import tpu_sc as plsc

N = 2048
D_IN = 768
D_OUT = 768
E = 64
T = 64
GL = 128

NC = 2
NS = 16
NW = NC * NS
BPW = N // NW


def _sc_mesh():
    return plsc.VectorSubcoreMesh(core_axis_name="c", subcore_axis_name="s",
                                  num_cores=NC, num_subcores=NS)


SCH = 4
CW = BPW // SCH


def _gather_body(inp_hbm, tok_hbm, g2_hbm, xs_hbm, gs_hbm,
                 idx_g, rows_g, *rest):
    idx_c = rest[0:SCH]
    rows_c = rest[SCH:2 * SCH]
    sem_st = rest[2 * SCH]
    sem_gx = rest[2 * SCH + 1]
    sem_wb = rest[2 * SCH + 2]
    sem_g = rest[2 * SCH + 3]
    wid = lax.axis_index("s") * NC + lax.axis_index("c")
    base = wid * BPW
    stg = [pltpu.async_copy(tok_hbm.at[pl.ds(base + j * CW, CW)], idx_c[j],
                            sem_st.at[j]) for j in range(SCH)]
    stg_g = pltpu.async_copy(tok_hbm.at[pl.ds(base, BPW)], idx_g, sem_g)
    gx = []
    for j in range(SCH):
        stg[j].wait()
        gx.append(pltpu.async_copy(inp_hbm.at[idx_c[j]], rows_c[j],
                                   sem_gx.at[j]))
    stg_g.wait()
    gg = pltpu.async_copy(g2_hbm.at[idx_g], rows_g, sem_g)
    wb = []
    for j in range(SCH):
        gx[j].wait()
        wb.append(pltpu.async_copy(rows_c[j],
                                   xs_hbm.at[pl.ds(base + j * CW, CW)],
                                   sem_wb.at[j]))
    gg.wait()
    wbg = pltpu.async_copy(rows_g, gs_hbm.at[pl.ds(base, BPW)], sem_g)
    for j in range(SCH):
        wb[j].wait()
    wbg.wait()


def _scatter_body(y_hbm, tok_hbm, out_hbm, *rest):
    idx_c = rest[0:SCH]
    rows_c = rest[SCH:2 * SCH]
    sem_st = rest[2 * SCH]
    sem_ld = rest[2 * SCH + 1]
    sem_sc = rest[2 * SCH + 2]
    wid = lax.axis_index("s") * NC + lax.axis_index("c")
    base = wid * BPW
    stg = [pltpu.async_copy(tok_hbm.at[pl.ds(base + j * CW, CW)], idx_c[j],
                            sem_st.at[j]) for j in range(SCH)]
    ld = [pltpu.async_copy(y_hbm.at[pl.ds(base + j * CW, CW)], rows_c[j],
                           sem_ld.at[j]) for j in range(SCH)]
    sc = []
    for j in range(SCH):
        stg[j].wait()
        ld[j].wait()
        sc.append(pltpu.async_copy(rows_c[j], out_hbm.at[idx_c[j]],
                                   sem_sc.at[j]))
    for j in range(SCH):
        sc[j].wait()


def _sc_gather(inputs, tok, g2):
    return pl.kernel(
        _gather_body,
        out_type=(jax.ShapeDtypeStruct((N, D_IN), jnp.float32),
                  jax.ShapeDtypeStruct((N, GL), jnp.float32)),
        mesh=_sc_mesh(),
        scratch_types=(
            [pltpu.VMEM((BPW,), jnp.int32),
             pltpu.VMEM((BPW, GL), jnp.float32)]
            + [pltpu.VMEM((CW,), jnp.int32) for _ in range(SCH)]
            + [pltpu.VMEM((CW, D_IN), jnp.float32) for _ in range(SCH)]
            + [pltpu.SemaphoreType.DMA((SCH,)),
               pltpu.SemaphoreType.DMA((SCH,)),
               pltpu.SemaphoreType.DMA((SCH,)),
               pltpu.SemaphoreType.DMA]
        ),
    )(inputs, tok, g2)


def _sc_scatter(y_sorted, tok):
    return pl.kernel(
        _scatter_body,
        out_type=jax.ShapeDtypeStruct((N, D_OUT), jnp.float32),
        mesh=_sc_mesh(),
        scratch_types=(
            [pltpu.VMEM((CW,), jnp.int32) for _ in range(SCH)]
            + [pltpu.VMEM((CW, D_OUT), jnp.float32) for _ in range(SCH)]
            + [pltpu.SemaphoreType.DMA((SCH,)),
               pltpu.SemaphoreType.DMA((SCH,)),
               pltpu.SemaphoreType.DMA((SCH,))]
        ),
    )(y_sorted, tok)


NBUF = 4


BLK = 128
NBLK = N // BLK


def _gemm_body(offs_ref, w_hbm, x_ref, g_ref, y_hbm, y_ref, wbuf, sems,
               sem_y):
    H = D_OUT // 2

    def flush_block(b):
        b = pl.multiple_of(b * BLK, BLK)
        pltpu.make_async_copy(y_ref.at[pl.ds(b, BLK)],
                              y_hbm.at[pl.ds(b, BLK)], sem_y).start()

    def start_fetch(e):
        b = lax.rem(e, NBUF)
        pltpu.make_async_copy(w_hbm.at[e, pl.ds(0, H)],
                              wbuf.at[b, pl.ds(0, H)], sems.at[b, 0]).start()
        pltpu.make_async_copy(w_hbm.at[e, pl.ds(H, H)],
                              wbuf.at[b, pl.ds(H, H)], sems.at[b, 1]).start()

    for e in range(NBUF):
        start_fetch(e)

    def step(e, nb):
        b = lax.rem(e, NBUF)
        pltpu.make_async_copy(w_hbm.at[e, pl.ds(0, H)],
                              wbuf.at[b, pl.ds(0, H)], sems.at[b, 0]).wait()
        pltpu.make_async_copy(w_hbm.at[e, pl.ds(H, H)],
                              wbuf.at[b, pl.ds(H, H)], sems.at[b, 1]).wait()
        s = jnp.where(e == 0, 0, offs_ref[jnp.maximum(e - 1, 0)])
        end = offs_ref[e]
        s8 = (s // 8) * 8
        nt = (end - s8 + T - 1) // T

        def body(i, _):
            base = pl.multiple_of(jnp.minimum(s8 + i * T, N - T), 8)
            xg = x_ref[pl.ds(base, T), :] * g_ref[pl.ds(base, T), 0:1]
            y = lax.dot_general(xg, wbuf[b],
                                dimension_numbers=(((1,), (1,)), ((), ())),
                                preferred_element_type=jnp.float32)
            q = base + lax.broadcasted_iota(jnp.int32, (T, D_OUT), 0)
            m = (q >= s) & (q < end)
            y_ref[pl.ds(base, T), :] = jnp.where(m, y,
                                                 y_ref[pl.ds(base, T), :])
            return 0

        lax.fori_loop(0, nt, body, 0)

        @pl.when(e + NBUF < E)
        def _():
            start_fetch(e + NBUF)

        done = end // BLK
        for _ in range(2):
            @pl.when(nb < done)
            def _():
                flush_block(nb)
            nb = jnp.where(nb < done, nb + 1, nb)
        return nb

    nb = lax.fori_loop(0, E, step, 0)
    for i in range(NBLK):
        @pl.when(i >= nb)
        def _():
            flush_block(jnp.int32(i))
    for _ in range(NBLK):
        pltpu.make_async_copy(y_ref.at[pl.ds(0, BLK)],
                              y_hbm.at[pl.ds(0, BLK)], sem_y).wait()


def _tc_grouped_gemm(expert_offsets, weight, x_sorted, g_sorted):
    return pl.pallas_call(
        _gemm_body,
        in_specs=[
            pl.BlockSpec(memory_space=pltpu.SMEM),
            pl.BlockSpec(memory_space=pltpu.MemorySpace.HBM),
            pl.BlockSpec(memory_space=pltpu.VMEM),
            pl.BlockSpec(memory_space=pltpu.VMEM),
        ],
        out_specs=pl.BlockSpec(memory_space=pltpu.MemorySpace.HBM),
        out_shape=jax.ShapeDtypeStruct((N, D_OUT), jnp.float32),
        scratch_shapes=[
            pltpu.VMEM((N, D_OUT), jnp.float32),
            pltpu.VMEM((NBUF, D_OUT, D_IN), jnp.float32),
            pltpu.SemaphoreType.DMA((NBUF, 2)),
            pltpu.SemaphoreType.DMA,
        ],
    )(expert_offsets, weight, x_sorted, g_sorted)


def kernel(inputs, weight, k, sorted_expert_idxs, sorted_scattered_idxs,
           expert_offsets, gates):
    tok = (sorted_scattered_idxs // k).astype(jnp.int32)
    g2 = jnp.broadcast_to(gates.reshape(N, 1).astype(jnp.float32), (N, GL))
    x_sorted, g_sorted = _sc_gather(inputs, tok, g2)
    y_sorted = _tc_grouped_gemm(expert_offsets, weight, x_sorted, g_sorted)
    return _sc_scatter(y_sorted, tok)
